# baseline (speedup 1.0000x reference)
import jax
import jax.numpy as jnp
from jax import lax
from jax.experimental import pallas as pl
from jax.experimental.pallas import tpu as pltpu
from jax.experimental.pallas import tpu_sc as plsc

EMBED = 32
BATCH = 16384
NW = 32
PER_W = BATCH // NW
CHUNK = 128
NCHUNK = PER_W // CHUNK
GROUPS = PER_W // 16


def _mf_body(user_h, item_p_h, item_n_h, users_t, items_t, out_p_h, out_n_h,
             idx_u, idx_p, idx_n, rows_u, rows_p, rows_n, out_p_v, out_n_v,
             sem):
    wid = lax.axis_index("s") * 2 + lax.axis_index("c")
    base = wid * PER_W

    cps = [
        pltpu.make_async_copy(user_h.at[pl.ds(base, PER_W)], idx_u, sem),
        pltpu.make_async_copy(item_p_h.at[pl.ds(base, PER_W)], idx_p, sem),
        pltpu.make_async_copy(item_n_h.at[pl.ds(base, PER_W)], idx_n, sem),
    ]
    for c in cps:
        c.start()
    for c in cps:
        c.wait()

    gathers = []
    for j in range(NCHUNK):
        sl = pl.ds(j * CHUNK, CHUNK)
        gathers.append(pltpu.make_async_copy(
            users_t.at[idx_u.at[sl]], rows_u.at[sl], sem))
        gathers.append(pltpu.make_async_copy(
            items_t.at[idx_p.at[sl]], rows_p.at[sl], sem))
        gathers.append(pltpu.make_async_copy(
            items_t.at[idx_n.at[sl]], rows_n.at[sl], sem))
    for g in gathers:
        g.start()
    for g in gathers:
        g.wait()

    lane = lax.iota(jnp.int32, 16)

    def group(g, carry):
        row = g * 16 + lane
        acc_p = jnp.zeros((16,), jnp.float32)
        acc_n = jnp.zeros((16,), jnp.float32)
        for d in range(EMBED):
            col = (lane + d) & (EMBED - 1)
            u = plsc.load_gather(rows_u, [row, col])
            p = plsc.load_gather(rows_p, [row, col])
            n = plsc.load_gather(rows_n, [row, col])
            acc_p = acc_p + u * p
            acc_n = acc_n + u * n
        out_p_v[pl.ds(g * 16, 16)] = acc_p
        out_n_v[pl.ds(g * 16, 16)] = acc_n
        return carry

    lax.fori_loop(0, GROUPS, group, 0)

    pltpu.sync_copy(out_p_v, out_p_h.at[pl.ds(base, PER_W)])
    pltpu.sync_copy(out_n_v, out_n_h.at[pl.ds(base, PER_W)])


@jax.jit
def _mf(user, item_p, item_n, users_table, items_table):
    mesh = plsc.VectorSubcoreMesh(core_axis_name="c", subcore_axis_name="s")
    f = pl.kernel(
        _mf_body,
        mesh=mesh,
        compiler_params=pltpu.CompilerParams(use_tc_tiling_on_sc=False,
                                             needs_layout_passes=False),
        out_type=(
            jax.ShapeDtypeStruct((BATCH,), jnp.float32),
            jax.ShapeDtypeStruct((BATCH,), jnp.float32),
        ),
        scratch_types=[
            pltpu.VMEM((PER_W,), jnp.int32),
            pltpu.VMEM((PER_W,), jnp.int32),
            pltpu.VMEM((PER_W,), jnp.int32),
            pltpu.VMEM((PER_W, EMBED), jnp.float32),
            pltpu.VMEM((PER_W, EMBED), jnp.float32),
            pltpu.VMEM((PER_W, EMBED), jnp.float32),
            pltpu.VMEM((PER_W,), jnp.float32),
            pltpu.VMEM((PER_W,), jnp.float32),
            pltpu.SemaphoreType.DMA,
        ],
    )
    return f(user, item_p, item_n, users_table, items_table)


def kernel(user, item_p, item_n, users_table, items_table):
    return _mf(user.astype(jnp.int32), item_p.astype(jnp.int32),
               item_n.astype(jnp.int32), users_table, items_table)
